# Initial kernel scaffold; baseline (speedup 1.0000x reference)
import jax
import jax.numpy as jnp
from jax.experimental import pallas as pl


def kernel(input1, input2):
    raise NotImplementedError("write your pallas kernel here")


# speedup vs baseline: 5.0120x; 5.0120x over previous
import jax
import jax.numpy as jnp
from jax import lax
from jax.experimental import pallas as pl
from jax.experimental.pallas import tpu as pltpu


def _crop_body(in_ref, out_ref, *, ms):
    x = in_ref[0]
    ch = x.shape[0] - 1
    h, w = x.shape[1], x.shape[2]
    occ = x[ch] != 0.0
    ridx = lax.broadcasted_iota(jnp.int32, (h, w), 0)
    cidx = lax.broadcasted_iota(jnp.int32, (h, w), 1)
    t0 = jnp.max(jnp.where(occ, ridx, -1))
    b0 = jnp.min(jnp.where(occ, ridx, h))
    t1 = jnp.max(jnp.where(occ, cidx, -1))
    b1 = jnp.min(jnp.where(occ, cidx, w))
    s0 = t0 - b0
    s1 = t1 - b1
    ti = (ms - s0) // 2
    li = (ms - s1) // 2
    dr = b0 - ti
    dc = b1 - li
    sh_r = (-dr) % h
    sh_c = (-dc) % w
    y2 = lax.broadcasted_iota(jnp.int32, (ms, ms), 0)
    x2 = lax.broadcasted_iota(jnp.int32, (ms, ms), 1)
    valid = (y2 >= ti) & (y2 < ti + s0) & (x2 >= li) & (x2 < li + s1)
    for c in range(ch):
        t = pltpu.roll(x[c], sh_r, 0)
        t = pltpu.roll(t, sh_c, 1)
        out_ref[0, c] = jnp.where(valid, t[:ms, :ms], 0.0)


def kernel(input1, input2):
    b, c4, h, w = input1.shape
    ch = c4 - 1
    ms = input2.shape[-1]
    import functools
    body = functools.partial(_crop_body, ms=ms)
    return pl.pallas_call(
        body,
        grid=(b,),
        in_specs=[pl.BlockSpec((1, c4, h, w), lambda i: (i, 0, 0, 0))],
        out_specs=pl.BlockSpec((1, ch, ms, ms), lambda i: (i, 0, 0, 0)),
        out_shape=jax.ShapeDtypeStruct((b, ch, ms, ms), input2.dtype),
    )(input1)
